# baseline (speedup 1.0000x reference)
import functools

import jax
import jax.numpy as jnp
from jax import lax
from jax.experimental import pallas as pl
from jax.experimental.pallas import tpu as pltpu
from jax.experimental.pallas import tpu_sc as plsc

N_ATOMS = 100000
N_IN = 128
N_HIDDEN = 64
N_MOL = 1024

LANES = 16
NWORKERS = 32
BLK = 12800
GRID_HALF = 4
HALF = GRID_HALF * BLK
N_PAD = 2 * HALF
CHUNK = HALF // NWORKERS
NVECS = CHUNK // LANES


def _mlp_body(x_ref, w1_ref, b1_ref, w2_ref, b2_ref, y_ref, *, base_blk):
    i = pl.program_id(0)
    h = jnp.dot(x_ref[...], w1_ref[...], preferred_element_type=jnp.float32)
    h = h + b1_ref[...]
    h = h * jax.nn.sigmoid(h)
    y = jnp.dot(h, w2_ref[...], preferred_element_type=jnp.float32) + b2_ref[...]
    rows = (base_blk + i) * BLK + lax.broadcasted_iota(jnp.int32, (BLK, 1), 0)
    y_ref[...] = jnp.where(rows < N_ATOMS, y, 0.0)


def _mlp_half(x, W1, b1r, W2, b2r, half):
    return pl.pallas_call(
        functools.partial(_mlp_body, base_blk=half * GRID_HALF),
        grid=(GRID_HALF,),
        in_specs=[
            pl.BlockSpec((BLK, N_IN), lambda i, h=half: (i + h * GRID_HALF, 0)),
            pl.BlockSpec((N_IN, N_HIDDEN), lambda i: (0, 0)),
            pl.BlockSpec((1, N_HIDDEN), lambda i: (0, 0)),
            pl.BlockSpec((N_HIDDEN, 1), lambda i: (0, 0)),
            pl.BlockSpec((1, 1), lambda i: (0, 0)),
        ],
        out_specs=pl.BlockSpec((BLK, 1), lambda i: (i, 0)),
        out_shape=jax.ShapeDtypeStruct((HALF, 1), jnp.float32),
    )(x, W1, b1r, W2, b2r)


def _sc_segment_sum(y_half, idx_pad, half):
    mesh = plsc.VectorSubcoreMesh(core_axis_name="c", subcore_axis_name="s")

    @functools.partial(
        pl.kernel,
        mesh=mesh,
        out_type=jax.ShapeDtypeStruct((NWORKERS, N_MOL), jnp.float32),
        scratch_types=[
            pltpu.VMEM((CHUNK,), jnp.float32),
            pltpu.VMEM((CHUNK,), jnp.int32),
            pltpu.VMEM((N_MOL,), jnp.float32),
        ],
        compiler_params=pltpu.CompilerParams(needs_layout_passes=False),
    )
    def body(y_hbm, idx_hbm, out_hbm, y_v, idx_v, acc_v):
        wid = lax.axis_index("s") * 2 + lax.axis_index("c")
        base = wid * CHUNK
        pltpu.sync_copy(y_hbm.at[pl.ds(base, CHUNK)], y_v)
        pltpu.sync_copy(idx_hbm.at[pl.ds(half * HALF + base, CHUNK)], idx_v)

        zero = jnp.zeros((LANES,), jnp.float32)

        def zero_body(k, _):
            acc_v[pl.ds(k * LANES, LANES)] = zero
            return 0

        lax.fori_loop(0, N_MOL // LANES, zero_body, 0)

        def scat_body(j, _):
            idx = idx_v[pl.ds(j * LANES, LANES)]
            val = y_v[pl.ds(j * LANES, LANES)]
            plsc.addupdate_scatter(acc_v, [idx], val)
            return 0

        lax.fori_loop(0, NVECS, scat_body, 0)
        pltpu.sync_copy(acc_v, out_hbm.at[wid])

    return body(y_half.reshape(HALF), idx_pad)


def _combine_body(p0_ref, p1_ref, o_ref):
    o_ref[...] = jnp.sum(p0_ref[...], axis=0, keepdims=True) + jnp.sum(
        p1_ref[...], axis=0, keepdims=True
    )


def _combine(p0, p1):
    return pl.pallas_call(
        _combine_body,
        out_shape=jax.ShapeDtypeStruct((1, N_MOL), jnp.float32),
    )(p0, p1)


def kernel(scalar_representation, idx_m, W1, b1, W2, b2):
    b1r = b1.reshape(1, N_HIDDEN)
    b2r = b2.reshape(1, 1)
    idx_pad = jnp.pad(idx_m.astype(jnp.int32), (0, N_PAD - N_ATOMS))
    y0 = _mlp_half(scalar_representation, W1, b1r, W2, b2r, 0)
    p0 = _sc_segment_sum(y0, idx_pad, 0)
    y1 = _mlp_half(scalar_representation, W1, b1r, W2, b2r, 1)
    p1 = _sc_segment_sum(y1, idx_pad, 1)
    out = _combine(p0, p1)
    return out.reshape(N_MOL)


# speedup vs baseline: 1.1499x; 1.1499x over previous
import functools

import jax
import jax.numpy as jnp
from jax import lax
from jax.experimental import pallas as pl
from jax.experimental.pallas import tpu as pltpu
from jax.experimental.pallas import tpu_sc as plsc

N_ATOMS = 100000
N_IN = 128
N_HIDDEN = 64
N_MOL = 1024

LANES = 16
NWORKERS = 32
BLK = 12800
GRID = 8
N_PAD = GRID * BLK
CHUNK = N_PAD // NWORKERS
NVECS = CHUNK // LANES


def _mlp_body(x_ref, w1_ref, b1_ref, w2_ref, b2_ref, y_ref):
    i = pl.program_id(0)
    h = jnp.dot(x_ref[...], w1_ref[...], preferred_element_type=jnp.float32)
    h = h + b1_ref[...]
    h = h * jax.nn.sigmoid(h)
    y = jnp.dot(h, w2_ref[...], preferred_element_type=jnp.float32) + b2_ref[...]
    rows = i * BLK + lax.broadcasted_iota(jnp.int32, (BLK, 1), 0)
    y_ref[...] = jnp.where(rows < N_ATOMS, y, 0.0)


def _mlp(x, W1, b1, W2, b2):
    return pl.pallas_call(
        _mlp_body,
        grid=(GRID,),
        in_specs=[
            pl.BlockSpec((BLK, N_IN), lambda i: (i, 0)),
            pl.BlockSpec((N_IN, N_HIDDEN), lambda i: (0, 0)),
            pl.BlockSpec((1, N_HIDDEN), lambda i: (0, 0)),
            pl.BlockSpec((N_HIDDEN, 1), lambda i: (0, 0)),
            pl.BlockSpec((1, 1), lambda i: (0, 0)),
        ],
        out_specs=pl.BlockSpec((BLK, 1), lambda i: (i, 0)),
        out_shape=jax.ShapeDtypeStruct((N_PAD, 1), jnp.float32),
    )(x, W1, b1.reshape(1, N_HIDDEN), W2, b2.reshape(1, 1))


def _sc_segment_sum(y_flat, idx_pad):
    mesh = plsc.VectorSubcoreMesh(core_axis_name="c", subcore_axis_name="s")

    @functools.partial(
        pl.kernel,
        mesh=mesh,
        out_type=jax.ShapeDtypeStruct((NWORKERS, N_MOL), jnp.float32),
        scratch_types=[
            pltpu.VMEM((CHUNK,), jnp.float32),
            pltpu.VMEM((CHUNK,), jnp.int32),
            pltpu.VMEM((N_MOL,), jnp.float32),
        ],
        compiler_params=pltpu.CompilerParams(needs_layout_passes=False),
    )
    def body(y_hbm, idx_hbm, out_hbm, y_v, idx_v, acc_v):
        wid = lax.axis_index("s") * 2 + lax.axis_index("c")
        base = wid * CHUNK
        pltpu.sync_copy(y_hbm.at[pl.ds(base, CHUNK)], y_v)
        pltpu.sync_copy(idx_hbm.at[pl.ds(base, CHUNK)], idx_v)

        zero = jnp.zeros((LANES,), jnp.float32)

        def zero_body(k, _):
            acc_v[pl.ds(k * LANES, LANES)] = zero
            return 0

        lax.fori_loop(0, N_MOL // LANES, zero_body, 0)

        def scat_body(j, _):
            idx = idx_v[pl.ds(j * LANES, LANES)]
            val = y_v[pl.ds(j * LANES, LANES)]
            plsc.addupdate_scatter(acc_v, [idx], val)
            return 0

        lax.fori_loop(0, NVECS, scat_body, 0)
        pltpu.sync_copy(acc_v, out_hbm.at[wid])

    return body(y_flat, idx_pad)


def _combine_body(p_ref, o_ref):
    o_ref[...] = jnp.sum(p_ref[...], axis=0, keepdims=True)


def _combine(partials):
    return pl.pallas_call(
        _combine_body,
        out_shape=jax.ShapeDtypeStruct((1, N_MOL), jnp.float32),
    )(partials)


def kernel(scalar_representation, idx_m, W1, b1, W2, b2):
    y_pad = _mlp(scalar_representation, W1, b1, W2, b2)
    y_flat = y_pad.reshape(N_PAD)
    idx32 = idx_m.astype(jnp.int32)
    idx_pad = jnp.pad(idx32, (0, N_PAD - N_ATOMS))
    partials = _sc_segment_sum(y_flat, idx_pad)
    out = _combine(partials)
    return out.reshape(N_MOL)


# speedup vs baseline: 1.7290x; 1.5037x over previous
import functools

import jax
import jax.numpy as jnp
from jax import lax
from jax.experimental import pallas as pl
from jax.experimental.pallas import tpu as pltpu
from jax.experimental.pallas import tpu_sc as plsc

N_ATOMS = 100000
N_IN = 128
N_HIDDEN = 64
N_MOL = 1024

LANES = 16
NWORKERS = 32
BLK = 10240
GRID = 10
N_PAD = GRID * BLK
CHUNK = N_PAD // NWORKERS
NVECS = CHUNK // LANES


ROWS = BLK // 128


def _mlp_body(x_ref, w1_ref, b1_ref, w2_ref, b2_ref, y_ref):
    i = pl.program_id(0)
    h = jnp.dot(x_ref[...], w1_ref[...], preferred_element_type=jnp.float32)
    h = h + b1_ref[...]
    h = h * jax.nn.sigmoid(h)
    y = jnp.dot(h, w2_ref[...], preferred_element_type=jnp.float32) + b2_ref[...]
    yw = y.reshape(ROWS, 128)
    rows = (
        i * BLK
        + lax.broadcasted_iota(jnp.int32, (ROWS, 128), 0) * 128
        + lax.broadcasted_iota(jnp.int32, (ROWS, 128), 1)
    )
    y_ref[...] = jnp.where(rows < N_ATOMS, yw, 0.0)


def _mlp(x, W1, b1, W2, b2):
    return pl.pallas_call(
        _mlp_body,
        grid=(GRID,),
        in_specs=[
            pl.BlockSpec((BLK, N_IN), lambda i: (i, 0)),
            pl.BlockSpec((N_IN, N_HIDDEN), lambda i: (0, 0)),
            pl.BlockSpec((1, N_HIDDEN), lambda i: (0, 0)),
            pl.BlockSpec((N_HIDDEN, 1), lambda i: (0, 0)),
            pl.BlockSpec((1, 1), lambda i: (0, 0)),
        ],
        out_specs=pl.BlockSpec((ROWS, 128), lambda i: (i, 0)),
        out_shape=jax.ShapeDtypeStruct((N_PAD // 128, 128), jnp.float32),
    )(x, W1, b1.reshape(1, N_HIDDEN), W2, b2.reshape(1, 1))


def _sc_segment_sum(y_flat, idx_pad):
    mesh = plsc.VectorSubcoreMesh(core_axis_name="c", subcore_axis_name="s")

    @functools.partial(
        pl.kernel,
        mesh=mesh,
        out_type=jax.ShapeDtypeStruct((NWORKERS, N_MOL), jnp.float32),
        scratch_types=[
            pltpu.VMEM((CHUNK,), jnp.float32),
            pltpu.VMEM((CHUNK,), jnp.int32),
            pltpu.VMEM((N_MOL,), jnp.float32),
        ],
        compiler_params=pltpu.CompilerParams(needs_layout_passes=False),
    )
    def body(y_hbm, idx_hbm, out_hbm, y_v, idx_v, acc_v):
        wid = lax.axis_index("s") * 2 + lax.axis_index("c")
        base = wid * CHUNK
        pltpu.sync_copy(y_hbm.at[pl.ds(base, CHUNK)], y_v)
        pltpu.sync_copy(idx_hbm.at[pl.ds(base, CHUNK)], idx_v)

        zero = jnp.zeros((LANES,), jnp.float32)

        def zero_body(k, _):
            acc_v[pl.ds(k * LANES, LANES)] = zero
            return 0

        lax.fori_loop(0, N_MOL // LANES, zero_body, 0)

        def scat_body(j, _):
            idx = idx_v[pl.ds(j * LANES, LANES)]
            val = y_v[pl.ds(j * LANES, LANES)]
            plsc.addupdate_scatter(acc_v, [idx], val)
            return 0

        lax.fori_loop(0, NVECS, scat_body, 0)
        pltpu.sync_copy(acc_v, out_hbm.at[wid])

    return body(y_flat, idx_pad)


def _combine_body(p_ref, o_ref):
    o_ref[...] = jnp.sum(p_ref[...], axis=0, keepdims=True)


def _combine(partials):
    return pl.pallas_call(
        _combine_body,
        out_shape=jax.ShapeDtypeStruct((1, N_MOL), jnp.float32),
    )(partials)


def kernel(scalar_representation, idx_m, W1, b1, W2, b2):
    y_pad = _mlp(scalar_representation, W1, b1, W2, b2)
    y_flat = y_pad.reshape(N_PAD)
    idx32 = idx_m.astype(jnp.int32)
    idx_pad = jnp.pad(idx32, (0, N_PAD - N_ATOMS))
    partials = _sc_segment_sum(y_flat, idx_pad)
    out = _combine(partials)
    return out.reshape(N_MOL)


# speedup vs baseline: 1.7913x; 1.0360x over previous
import functools

import jax
import jax.numpy as jnp
from jax import lax
from jax.experimental import pallas as pl
from jax.experimental.pallas import tpu as pltpu
from jax.experimental.pallas import tpu_sc as plsc

N_ATOMS = 100000
N_IN = 128
N_HIDDEN = 64
N_MOL = 1024

LANES = 16
NWORKERS = 32
BLK = 10240
GRID = 10
N_PAD = GRID * BLK
CHUNK = N_PAD // NWORKERS
NVECS = CHUNK // LANES
TAIL = N_ATOMS - (NWORKERS - 1) * CHUNK


ROWS = BLK // 128


def _mlp_body(x_ref, w1_ref, b1_ref, w2_ref, b2_ref, y_ref):
    i = pl.program_id(0)
    h = jnp.dot(x_ref[...], w1_ref[...], preferred_element_type=jnp.float32)
    h = h + b1_ref[...]
    h = h * jax.nn.sigmoid(h)
    y = jnp.dot(h, w2_ref[...], preferred_element_type=jnp.float32) + b2_ref[...]
    yw = y.reshape(ROWS, 128)
    rows = (
        i * BLK
        + lax.broadcasted_iota(jnp.int32, (ROWS, 128), 0) * 128
        + lax.broadcasted_iota(jnp.int32, (ROWS, 128), 1)
    )
    y_ref[...] = jnp.where(rows < N_ATOMS, yw, 0.0)


def _mlp(x, W1, b1, W2, b2):
    return pl.pallas_call(
        _mlp_body,
        grid=(GRID,),
        in_specs=[
            pl.BlockSpec((BLK, N_IN), lambda i: (i, 0)),
            pl.BlockSpec((N_IN, N_HIDDEN), lambda i: (0, 0)),
            pl.BlockSpec((1, N_HIDDEN), lambda i: (0, 0)),
            pl.BlockSpec((N_HIDDEN, 1), lambda i: (0, 0)),
            pl.BlockSpec((1, 1), lambda i: (0, 0)),
        ],
        out_specs=pl.BlockSpec((ROWS, 128), lambda i: (i, 0)),
        out_shape=jax.ShapeDtypeStruct((N_PAD // 128, 128), jnp.float32),
    )(x, W1, b1.reshape(1, N_HIDDEN), W2, b2.reshape(1, 1))


def _sc_segment_sum(y_flat, idx_pad):
    mesh = plsc.VectorSubcoreMesh(core_axis_name="c", subcore_axis_name="s")

    @functools.partial(
        pl.kernel,
        mesh=mesh,
        out_type=jax.ShapeDtypeStruct((NWORKERS, N_MOL), jnp.float32),
        scratch_types=[
            pltpu.VMEM((CHUNK,), jnp.float32),
            pltpu.VMEM((CHUNK,), jnp.int32),
            pltpu.VMEM((N_MOL,), jnp.float32),
        ],
        compiler_params=pltpu.CompilerParams(needs_layout_passes=False),
    )
    def body(y_hbm, idx_hbm, out_hbm, y_v, idx_v, acc_v):
        wid = lax.axis_index("s") * 2 + lax.axis_index("c")
        base = wid * CHUNK
        pltpu.sync_copy(y_hbm.at[pl.ds(base, CHUNK)], y_v)

        @pl.when(wid < NWORKERS - 1)
        def _():
            pltpu.sync_copy(idx_hbm.at[pl.ds(base, CHUNK)], idx_v)

        @pl.when(wid == NWORKERS - 1)
        def _():
            pltpu.sync_copy(
                idx_hbm.at[pl.ds((NWORKERS - 1) * CHUNK, TAIL)],
                idx_v.at[pl.ds(0, TAIL)],
            )

        zero = jnp.zeros((LANES,), jnp.float32)

        def zero_body(k, _):
            acc_v[pl.ds(k * LANES, LANES)] = zero
            return 0

        lax.fori_loop(0, N_MOL // LANES, zero_body, 0)

        def scat_body(j, _):
            idx = idx_v[pl.ds(j * LANES, LANES)]
            val = y_v[pl.ds(j * LANES, LANES)]
            plsc.addupdate_scatter(acc_v, [idx], val)
            return 0

        nvecs = jnp.where(wid == NWORKERS - 1, TAIL // LANES, NVECS)
        lax.fori_loop(0, nvecs, scat_body, 0)
        pltpu.sync_copy(acc_v, out_hbm.at[wid])

    return body(y_flat, idx_pad)


def _combine_body(p_ref, o_ref):
    o_ref[...] = jnp.sum(p_ref[...], axis=0, keepdims=True)


def _combine(partials):
    return pl.pallas_call(
        _combine_body,
        out_shape=jax.ShapeDtypeStruct((1, N_MOL), jnp.float32),
    )(partials)


def kernel(scalar_representation, idx_m, W1, b1, W2, b2):
    y_pad = _mlp(scalar_representation, W1, b1, W2, b2)
    y_flat = y_pad.reshape(N_PAD)
    partials = _sc_segment_sum(y_flat, idx_m.astype(jnp.int32))
    out = _combine(partials)
    return out.reshape(N_MOL)


# speedup vs baseline: 1.8350x; 1.0244x over previous
import functools

import jax
import jax.numpy as jnp
from jax import lax
from jax.experimental import pallas as pl
from jax.experimental.pallas import tpu as pltpu
from jax.experimental.pallas import tpu_sc as plsc

N_ATOMS = 100000
N_IN = 128
N_HIDDEN = 64
N_MOL = 1024

LANES = 16
NWORKERS = 32
BLK = 20480
GRID = 5
N_PAD = GRID * BLK
CHUNK = N_PAD // NWORKERS
NVECS = CHUNK // LANES
TAIL = N_ATOMS - (NWORKERS - 1) * CHUNK


ROWS = BLK // 128


def _mlp_body(x_ref, w1_ref, b1_ref, w2_ref, b2_ref, y_ref):
    i = pl.program_id(0)
    h = jnp.dot(x_ref[...], w1_ref[...], preferred_element_type=jnp.float32)
    h = h + b1_ref[...]
    h = h * jax.nn.sigmoid(h)
    y = jnp.dot(h, w2_ref[...], preferred_element_type=jnp.float32) + b2_ref[...]
    yw = y.reshape(ROWS, 128)
    rows = (
        i * BLK
        + lax.broadcasted_iota(jnp.int32, (ROWS, 128), 0) * 128
        + lax.broadcasted_iota(jnp.int32, (ROWS, 128), 1)
    )
    y_ref[...] = jnp.where(rows < N_ATOMS, yw, 0.0)


def _mlp(x, W1, b1, W2, b2):
    return pl.pallas_call(
        _mlp_body,
        grid=(GRID,),
        in_specs=[
            pl.BlockSpec((BLK, N_IN), lambda i: (i, 0)),
            pl.BlockSpec((N_IN, N_HIDDEN), lambda i: (0, 0)),
            pl.BlockSpec((1, N_HIDDEN), lambda i: (0, 0)),
            pl.BlockSpec((N_HIDDEN, 1), lambda i: (0, 0)),
            pl.BlockSpec((1, 1), lambda i: (0, 0)),
        ],
        out_specs=pl.BlockSpec((ROWS, 128), lambda i: (i, 0)),
        out_shape=jax.ShapeDtypeStruct((N_PAD // 128, 128), jnp.float32),
    )(x, W1, b1.reshape(1, N_HIDDEN), W2, b2.reshape(1, 1))


def _sc_segment_sum(y_flat, idx_pad):
    mesh = plsc.VectorSubcoreMesh(core_axis_name="c", subcore_axis_name="s")

    @functools.partial(
        pl.kernel,
        mesh=mesh,
        out_type=jax.ShapeDtypeStruct((NWORKERS, N_MOL), jnp.float32),
        scratch_types=[
            pltpu.VMEM((CHUNK,), jnp.float32),
            pltpu.VMEM((CHUNK,), jnp.int32),
            pltpu.VMEM((N_MOL,), jnp.float32),
        ],
        compiler_params=pltpu.CompilerParams(needs_layout_passes=False),
    )
    def body(y_hbm, idx_hbm, out_hbm, y_v, idx_v, acc_v):
        wid = lax.axis_index("s") * 2 + lax.axis_index("c")
        base = wid * CHUNK
        pltpu.sync_copy(y_hbm.at[pl.ds(base, CHUNK)], y_v)

        @pl.when(wid < NWORKERS - 1)
        def _():
            pltpu.sync_copy(idx_hbm.at[pl.ds(base, CHUNK)], idx_v)

        @pl.when(wid == NWORKERS - 1)
        def _():
            pltpu.sync_copy(
                idx_hbm.at[pl.ds((NWORKERS - 1) * CHUNK, TAIL)],
                idx_v.at[pl.ds(0, TAIL)],
            )

        zero = jnp.zeros((LANES,), jnp.float32)

        def zero_body(k, _):
            acc_v[pl.ds(k * LANES, LANES)] = zero
            return 0

        lax.fori_loop(0, N_MOL // LANES, zero_body, 0)

        def scat_body(j, _):
            idx = idx_v[pl.ds(j * LANES, LANES)]
            val = y_v[pl.ds(j * LANES, LANES)]
            plsc.addupdate_scatter(acc_v, [idx], val)
            return 0

        nvecs = jnp.where(wid == NWORKERS - 1, TAIL // LANES, NVECS)
        lax.fori_loop(0, nvecs, scat_body, 0)
        pltpu.sync_copy(acc_v, out_hbm.at[wid])

    return body(y_flat, idx_pad)


def _combine_body(p_ref, o_ref):
    o_ref[...] = jnp.sum(p_ref[...], axis=0, keepdims=True)


def _combine(partials):
    return pl.pallas_call(
        _combine_body,
        out_shape=jax.ShapeDtypeStruct((1, N_MOL), jnp.float32),
    )(partials)


def kernel(scalar_representation, idx_m, W1, b1, W2, b2):
    y_pad = _mlp(scalar_representation, W1, b1, W2, b2)
    y_flat = y_pad.reshape(N_PAD)
    partials = _sc_segment_sum(y_flat, idx_m.astype(jnp.int32))
    out = _combine(partials)
    return out.reshape(N_MOL)
